# Initial kernel scaffold; baseline (speedup 1.0000x reference)
import jax
import jax.numpy as jnp
from jax.experimental import pallas as pl


def kernel(points, new_points, features):
    raise NotImplementedError("write your pallas kernel here")


# speedup vs baseline: 4.5355x; 4.5355x over previous
import functools

import jax
import jax.numpy as jnp
from jax import lax
from jax.experimental import pallas as pl
from jax.experimental.pallas import tpu as pltpu
from jax.experimental.pallas import tpu_sc as plsc

S = 32
D = 144
NW = 32
NC = 2


def _table_body(f_ref, p_ref, o_ref):
    f = f_ref[0]
    p = p_ref[0]
    o_ref[0, :, 0:128] = f.T
    o_ref[0, :, 128:131] = p.T


def _topk_body(p_ref, q_ref, o_ref, dist_ref, *, n, bm, nc):
    b = pl.program_id(0)
    q = q_ref[0]
    yy = jnp.sum(q * q, axis=0)
    nch = n // nc
    inf = jnp.float32(jnp.inf)
    nn = jnp.int32(n)

    def dist_chunk(i, carry):
        pc = p_ref[0, :, pl.ds(i * nc, nc)]
        inner = lax.dot_general(pc, q, (((0,), (0,)), ((), ())),
                                preferred_element_type=jnp.float32)
        xx = jnp.sum(pc * pc, axis=0)
        d = xx[:, None] - 2.0 * inner + yy[None, :]
        dist_ref[pl.ds(i * nc, nc), :] = d
        m = jnp.min(d, axis=0)
        return jnp.minimum(carry, m)

    mn = lax.fori_loop(0, nch, dist_chunk, jnp.full((bm,), inf), unroll=False)

    def argmin_chunk(i, carry):
        best, cur_mn = carry
        d = dist_ref[pl.ds(i * nc, nc), :]
        iot = lax.broadcasted_iota(jnp.int32, (nc, bm), 0) + i * nc
        cand = jnp.min(jnp.where(d == cur_mn[None, :], iot, nn), axis=0)
        return jnp.minimum(best, cand), cur_mn

    def mask_and_min_chunk(i, carry):
        idx, macc = carry
        d = dist_ref[pl.ds(i * nc, nc), :]
        iot = lax.broadcasted_iota(jnp.int32, (nc, bm), 0) + i * nc
        d = jnp.where(iot == idx[None, :], inf, d)
        dist_ref[pl.ds(i * nc, nc), :] = d
        return idx, jnp.minimum(macc, jnp.min(d, axis=0))

    cols = []
    for _ in range(S):
        idx, _ = lax.fori_loop(0, nch, argmin_chunk,
                               (jnp.full((bm,), nn), mn), unroll=False)
        cols.append(idx[:, None])
        _, mn = lax.fori_loop(0, nch, mask_and_min_chunk,
                              (idx, jnp.full((bm,), inf)), unroll=False)
    o_ref[0] = jnp.concatenate(cols, axis=1) + b * nn


def _gather_body(tab_ref, idx_ref, out_ref, idx_v, rows_v, sem,
                 *, rows_per_w, chunk):
    wid = lax.axis_index("s") * NC + lax.axis_index("c")
    base = wid * rows_per_w

    def step(i, carry):
        off = base + i * chunk
        pltpu.sync_copy(idx_ref.at[pl.ds(off, chunk)], idx_v)
        pltpu.async_copy(tab_ref.at[idx_v], rows_v, sem).wait()
        pltpu.sync_copy(rows_v, out_ref.at[pl.ds(off, chunk)])
        return carry

    lax.fori_loop(0, rows_per_w // chunk, step, 0)


def _finish_body(r_ref, q_ref, o_ref):
    t = r_ref[0].T
    o_ref[0, 0:3, :] = t[128:131, :] - q_ref[0]
    o_ref[0, 3:131, :] = t[0:128, :]


def kernel(points, new_points, features):
    B, three, N = points.shape
    M = new_points.shape[2]
    C = features.shape[1]

    NB = 1024
    table = pl.pallas_call(
        _table_body,
        grid=(B, N // NB),
        in_specs=[
            pl.BlockSpec((1, C, NB), lambda b, j: (b, 0, j)),
            pl.BlockSpec((1, three, NB), lambda b, j: (b, 0, j)),
        ],
        out_specs=pl.BlockSpec((1, NB, D), lambda b, j: (b, j, 0)),
        out_shape=jax.ShapeDtypeStruct((B, N, D), jnp.float32),
    )(features, points)
    table = table.reshape(B * N, D)

    BM = 128
    NCHUNK = 256
    ind = pl.pallas_call(
        functools.partial(_topk_body, n=N, bm=BM, nc=NCHUNK),
        grid=(B, M // BM),
        in_specs=[
            pl.BlockSpec((1, three, N), lambda b, j: (b, 0, 0)),
            pl.BlockSpec((1, three, BM), lambda b, j: (b, 0, j)),
        ],
        out_specs=pl.BlockSpec((1, BM, S), lambda b, j: (b, j, 0)),
        out_shape=jax.ShapeDtypeStruct((B, M, S), jnp.int32),
        scratch_shapes=[pltpu.VMEM((N, BM), jnp.float32)],
    )(points, new_points)
    idx_flat = ind.reshape(B * M * S)

    RT = B * M * S
    rows_per_w = RT // NW
    chunk = 128
    mesh = plsc.VectorSubcoreMesh(core_axis_name="c", subcore_axis_name="s")
    rows = pl.kernel(
        functools.partial(_gather_body, rows_per_w=rows_per_w, chunk=chunk),
        out_type=jax.ShapeDtypeStruct((RT, D), jnp.float32),
        mesh=mesh,
        scratch_types=[
            pltpu.VMEM((chunk,), jnp.int32),
            pltpu.VMEM((chunk, D), jnp.float32),
            pltpu.SemaphoreType.DMA,
        ],
        compiler_params=pltpu.CompilerParams(use_tc_tiling_on_sc=False),
    )(table, idx_flat)

    BMC = 128
    BMS = BMC * S
    qrep = jnp.broadcast_to(new_points[:, :, :, None],
                            (B, three, M, S)).reshape(B, three, M * S)
    rows3 = rows.reshape(B, M * S, D)
    out = pl.pallas_call(
        _finish_body,
        grid=(B, (M * S) // BMS),
        in_specs=[
            pl.BlockSpec((1, BMS, D), lambda b, j: (b, j, 0)),
            pl.BlockSpec((1, three, BMS), lambda b, j: (b, 0, j)),
        ],
        out_specs=pl.BlockSpec((1, C + three, BMS), lambda b, j: (b, 0, j)),
        out_shape=jax.ShapeDtypeStruct((B, C + three, M * S), jnp.float32),
    )(rows3, qrep)
    return out.reshape(B, C + three, M, S)


# speedup vs baseline: 6.2425x; 1.3764x over previous
import functools

import jax
import jax.numpy as jnp
from jax import lax
from jax.experimental import pallas as pl
from jax.experimental.pallas import tpu as pltpu
from jax.experimental.pallas import tpu_sc as plsc

S = 32
D = 144
NW = 32
NC = 2
L = 16


def _table_body(f_ref, p_ref, o_ref):
    f = f_ref[0]
    p = p_ref[0]
    o_ref[0, :, 0:128] = f.T
    o_ref[0, :, 128:131] = p.T


def _dist_body(p_ref, q_ref, o_ref, *, n, nc):
    q = q_ref[0]
    yy = jnp.sum(q * q, axis=0)

    def chunk(i, c):
        pc = p_ref[0, :, pl.ds(i * nc, nc)]
        inner = lax.dot_general(pc, q, (((0,), (0,)), ((), ())),
                                preferred_element_type=jnp.float32)
        xx = jnp.sum(pc * pc, axis=0)
        d = xx[:, None] - 2.0 * inner + yy[None, :]
        o_ref[0, :, pl.ds(i * nc, nc)] = d.T
        return c

    lax.fori_loop(0, n // nc, chunk, 0, unroll=False)


def _xreduce(v, op):
    vals = [v[i] for i in range(L)]
    while len(vals) > 1:
        vals = [op(vals[i], vals[i + 1]) if i + 1 < len(vals) else vals[i]
                for i in range(0, len(vals), 2)]
    return vals[0]


def _sctopk_body(dist_ref, idx_ref, row_v, smin_ref, obuf, sem,
                 *, n, m, qpw):
    wid = lax.axis_index("s") * NC + lax.axis_index("c")
    qbase = wid * qpw
    inf = jnp.float32(jnp.inf)
    nch = n // L

    iot0 = lax.iota(jnp.int32, L)
    inf16 = jnp.full((L,), inf, jnp.float32)
    big = jnp.int32(1 << 30)
    big16 = jnp.full((L,), big, jnp.int32)
    G = 16
    nst = nch // G

    def per_query(j, carry0):
        qid = qbase + j
        b = qid // m
        pltpu.async_copy(dist_ref.at[qid], row_v, sem).wait()

        def p1(s, c):
            def inner(i, acc):
                return jnp.minimum(acc, row_v[pl.ds((s * G + i) * L, L)])
            smin_ref[pl.ds(s * L, L)] = lax.fori_loop(0, G, inner, inf16,
                                                      unroll=False)
            return c

        lax.fori_loop(0, nst, p1, 0, unroll=False)

        def extract(si, carry):
            def l1(s, acc):
                return jnp.minimum(acc, smin_ref[pl.ds(s * L, L)])
            macc = lax.fori_loop(0, nst, l1, inf16, unroll=False)
            gmin = _xreduce(macc, jnp.minimum)

            def l2(s, acc):
                eq = smin_ref[pl.ds(s * L, L)] == gmin
                return jnp.minimum(acc, jnp.where(eq, s, big))
            sstar = _xreduce(
                lax.fori_loop(0, nst, l2, big16, unroll=False), jnp.minimum)

            def l3(i, acc):
                base_n = (sstar * G + i) * L
                v = row_v[pl.ds(base_n, L)]
                eq = v == gmin
                return jnp.minimum(acc, jnp.where(eq, base_n + iot0, big))
            nsel = _xreduce(
                lax.fori_loop(0, G, l3, big16, unroll=False), jnp.minimum)

            cbase = (nsel >> 4) * L
            v = row_v[pl.ds(cbase, L)]
            row_v[pl.ds(cbase, L)] = jnp.where(iot0 == (nsel & (L - 1)),
                                               inf, v)

            def l4(i, acc):
                return jnp.minimum(acc, row_v[pl.ds((sstar * G + i) * L, L)])
            smin_ref[pl.ds(sstar * L, L)] = lax.fori_loop(0, G, l4, inf16,
                                                          unroll=False)

            ob = obuf[pl.ds((si >> 4) * L, L)]
            obuf[pl.ds((si >> 4) * L, L)] = jnp.where(
                iot0 == (si & (L - 1)), b * n + nsel, ob)
            return carry

        lax.fori_loop(0, S, extract, 0, unroll=False)
        pltpu.sync_copy(obuf, idx_ref.at[pl.ds(qid * S, S)])
        return carry0

    lax.fori_loop(0, qpw, per_query, 0, unroll=False)


def _gather_body(tab_ref, idx_ref, out_ref, idx_v, rows_v, sem,
                 *, rows_per_w, chunk):
    wid = lax.axis_index("s") * NC + lax.axis_index("c")
    base = wid * rows_per_w

    def step(i, carry):
        off = base + i * chunk
        pltpu.sync_copy(idx_ref.at[pl.ds(off, chunk)], idx_v)
        pltpu.async_copy(tab_ref.at[idx_v], rows_v, sem).wait()
        pltpu.sync_copy(rows_v, out_ref.at[pl.ds(off, chunk)])
        return carry

    lax.fori_loop(0, rows_per_w // chunk, step, 0)


def _finish_body(r_ref, q_ref, o_ref):
    t = r_ref[0].T
    o_ref[0, 0:3, :] = t[128:131, :] - q_ref[0]
    o_ref[0, 3:131, :] = t[0:128, :]


def kernel(points, new_points, features):
    B, three, N = points.shape
    M = new_points.shape[2]
    C = features.shape[1]

    NB = 1024
    table = pl.pallas_call(
        _table_body,
        grid=(B, N // NB),
        in_specs=[
            pl.BlockSpec((1, C, NB), lambda b, j: (b, 0, j)),
            pl.BlockSpec((1, three, NB), lambda b, j: (b, 0, j)),
        ],
        out_specs=pl.BlockSpec((1, NB, D), lambda b, j: (b, j, 0)),
        out_shape=jax.ShapeDtypeStruct((B, N, D), jnp.float32),
    )(features, points)
    table = table.reshape(B * N, D)

    BM = 256
    NCHUNK = 256
    dist_t = pl.pallas_call(
        functools.partial(_dist_body, n=N, nc=NCHUNK),
        grid=(B, M // BM),
        in_specs=[
            pl.BlockSpec((1, three, N), lambda b, j: (b, 0, 0)),
            pl.BlockSpec((1, three, BM), lambda b, j: (b, 0, j)),
        ],
        out_specs=pl.BlockSpec((1, BM, N), lambda b, j: (b, j, 0)),
        out_shape=jax.ShapeDtypeStruct((B, M, N), jnp.float32),
    )(points, new_points)
    dist_flat = dist_t.reshape(B * M, N)

    QT = B * M
    mesh = plsc.VectorSubcoreMesh(core_axis_name="c", subcore_axis_name="s")
    idx_flat = pl.kernel(
        functools.partial(_sctopk_body, n=N, m=M, qpw=QT // NW),
        out_type=jax.ShapeDtypeStruct((QT * S,), jnp.int32),
        mesh=mesh,
        scratch_types=[
            pltpu.VMEM((N,), jnp.float32),
            pltpu.VMEM((N // 16,), jnp.float32),
            pltpu.VMEM((S,), jnp.int32),
            pltpu.SemaphoreType.DMA,
        ],
        compiler_params=pltpu.CompilerParams(use_tc_tiling_on_sc=False),
    )(dist_flat)

    RT = B * M * S
    rows_per_w = RT // NW
    chunk = 128
    rows = pl.kernel(
        functools.partial(_gather_body, rows_per_w=rows_per_w, chunk=chunk),
        out_type=jax.ShapeDtypeStruct((RT, D), jnp.float32),
        mesh=mesh,
        scratch_types=[
            pltpu.VMEM((chunk,), jnp.int32),
            pltpu.VMEM((chunk, D), jnp.float32),
            pltpu.SemaphoreType.DMA,
        ],
        compiler_params=pltpu.CompilerParams(use_tc_tiling_on_sc=False),
    )(table, idx_flat)

    BMC = 128
    BMS = BMC * S
    qrep = jnp.broadcast_to(new_points[:, :, :, None],
                            (B, three, M, S)).reshape(B, three, M * S)
    rows3 = rows.reshape(B, M * S, D)
    out = pl.pallas_call(
        _finish_body,
        grid=(B, (M * S) // BMS),
        in_specs=[
            pl.BlockSpec((1, BMS, D), lambda b, j: (b, j, 0)),
            pl.BlockSpec((1, three, BMS), lambda b, j: (b, 0, j)),
        ],
        out_specs=pl.BlockSpec((1, C + three, BMS), lambda b, j: (b, 0, j)),
        out_shape=jax.ShapeDtypeStruct((B, C + three, M * S), jnp.float32),
    )(rows3, qrep)
    return out.reshape(B, C + three, M, S)


# speedup vs baseline: 10.5297x; 1.6868x over previous
import functools

import jax
import jax.numpy as jnp
from jax import lax
from jax.experimental import pallas as pl
from jax.experimental.pallas import tpu as pltpu
from jax.experimental.pallas import tpu_sc as plsc

S = 32
D = 144
NW = 32
NC = 2
L = 16


def _table_body(f_ref, p_ref, o_ref):
    f = f_ref[0]
    p = p_ref[0]
    o_ref[0, :, 0:128] = f.T
    o_ref[0, :, 128:131] = p.T


def _dist_body(p_ref, q_ref, o_ref, *, n, nc):
    q = q_ref[0]
    yy = jnp.sum(q * q, axis=0)

    def chunk(i, c):
        pc = p_ref[0, :, pl.ds(i * nc, nc)]
        inner = lax.dot_general(pc, q, (((0,), (0,)), ((), ())),
                                preferred_element_type=jnp.float32)
        xx = jnp.sum(pc * pc, axis=0)
        d = xx[:, None] - 2.0 * inner + yy[None, :]
        o_ref[0, :, pl.ds(i * nc, nc)] = d.T
        return c

    lax.fori_loop(0, n // nc, chunk, 0, unroll=False)


def _xreduce(v, op):
    vals = [v[i] for i in range(L)]
    while len(vals) > 1:
        vals = [op(vals[i], vals[i + 1]) if i + 1 < len(vals) else vals[i]
                for i in range(0, len(vals), 2)]
    return vals[0]


def _sctopk_body(dist_ref, idx_ref, row2, smin_ref, obuf, sem0, sem1,
                 *, n, m, qpw):
    wid = lax.axis_index("s") * NC + lax.axis_index("c")
    qbase = wid * qpw
    inf = jnp.float32(jnp.inf)
    nch = n // L

    iot0 = lax.iota(jnp.int32, L)
    inf16 = jnp.full((L,), inf, jnp.float32)
    big = jnp.int32(1 << 30)
    big16 = jnp.full((L,), big, jnp.int32)
    G = 16
    nst = nch // G

    pltpu.async_copy(dist_ref.at[qbase], row2.at[pl.ds(0, n)], sem0)

    def per_query(j, carry0):
        qid = qbase + j
        b = qid // m
        par = (j & 1) * n
        nxt = ((j + 1) & 1) * n

        @pl.when(((j & 1) == 0) & (j + 1 < qpw))
        def _():
            pltpu.async_copy(dist_ref.at[qid + 1], row2.at[pl.ds(nxt, n)],
                             sem1)

        @pl.when(((j & 1) == 1) & (j + 1 < qpw))
        def _():
            pltpu.async_copy(dist_ref.at[qid + 1], row2.at[pl.ds(nxt, n)],
                             sem0)

        @pl.when((j & 1) == 0)
        def _():
            pltpu.make_async_copy(dist_ref.at[qid], row2.at[pl.ds(par, n)],
                                  sem0).wait()

        @pl.when((j & 1) == 1)
        def _():
            pltpu.make_async_copy(dist_ref.at[qid], row2.at[pl.ds(par, n)],
                                  sem1).wait()

        def p1(s, c):
            def inner(i, acc):
                return jnp.minimum(acc, row2[pl.ds(par + (s * G + i) * L, L)])
            smin_ref[pl.ds(s * L, L)] = lax.fori_loop(0, G, inner, inf16,
                                                      unroll=16)
            return c

        lax.fori_loop(0, nst, p1, 0, unroll=4)

        def extract(si, carry):
            def l12(s, c):
                macc, sarg = c
                v = smin_ref[pl.ds(s * L, L)]
                upd = v < macc
                sarg = jnp.where(upd, s, sarg)
                macc = jnp.minimum(macc, v)
                return macc, sarg

            macc, sarg = lax.fori_loop(0, nst, l12, (inf16, big16), unroll=8)
            gmin = _xreduce(macc, jnp.minimum)
            sstar = _xreduce(jnp.where(macc == gmin, sarg, big16),
                             jnp.minimum)

            def l34(i, c):
                nmin, m1, m2 = c
                base_n = (sstar * G + i) * L
                v = row2[pl.ds(par + base_n, L)]
                eq = v == gmin
                nmin = jnp.minimum(nmin, jnp.where(eq, base_n + iot0, big))
                m2 = jnp.minimum(m2, jnp.maximum(m1, v))
                m1 = jnp.minimum(m1, v)
                return nmin, m1, m2

            nminv, m1, m2 = lax.fori_loop(0, G, l34, (big16, inf16, inf16),
                                          unroll=16)
            nsel = _xreduce(nminv, jnp.minimum)
            lane_sel = iot0 == (nsel & (L - 1))

            cbase = (nsel >> 4) * L
            v = row2[pl.ds(par + cbase, L)]
            row2[pl.ds(par + cbase, L)] = jnp.where(lane_sel, inf, v)

            smin_ref[pl.ds(sstar * L, L)] = jnp.where(lane_sel, m2, m1)

            ob = obuf[pl.ds((si >> 4) * L, L)]
            obuf[pl.ds((si >> 4) * L, L)] = jnp.where(
                iot0 == (si & (L - 1)), b * n + nsel, ob)
            return carry

        lax.fori_loop(0, S, extract, 0, unroll=False)
        pltpu.sync_copy(obuf, idx_ref.at[pl.ds(qid * S, S)])
        return carry0

    lax.fori_loop(0, qpw, per_query, 0, unroll=False)


def _gather_body(tab_ref, idx_ref, out_ref, idx_v, rows_v, sem,
                 *, rows_per_w, chunk):
    wid = lax.axis_index("s") * NC + lax.axis_index("c")
    base = wid * rows_per_w

    def step(i, carry):
        off = base + i * chunk
        pltpu.sync_copy(idx_ref.at[pl.ds(off, chunk)], idx_v)
        pltpu.async_copy(tab_ref.at[idx_v], rows_v, sem).wait()
        pltpu.sync_copy(rows_v, out_ref.at[pl.ds(off, chunk)])
        return carry

    lax.fori_loop(0, rows_per_w // chunk, step, 0)


def _finish_body(r_ref, q_ref, o_ref):
    t = r_ref[0].T
    o_ref[0, 0:3, :] = t[128:131, :] - q_ref[0]
    o_ref[0, 3:131, :] = t[0:128, :]


def kernel(points, new_points, features):
    B, three, N = points.shape
    M = new_points.shape[2]
    C = features.shape[1]

    NB = 1024
    table = pl.pallas_call(
        _table_body,
        grid=(B, N // NB),
        in_specs=[
            pl.BlockSpec((1, C, NB), lambda b, j: (b, 0, j)),
            pl.BlockSpec((1, three, NB), lambda b, j: (b, 0, j)),
        ],
        out_specs=pl.BlockSpec((1, NB, D), lambda b, j: (b, j, 0)),
        out_shape=jax.ShapeDtypeStruct((B, N, D), jnp.float32),
    )(features, points)
    table = table.reshape(B * N, D)

    BM = 256
    NCHUNK = 256
    dist_t = pl.pallas_call(
        functools.partial(_dist_body, n=N, nc=NCHUNK),
        grid=(B, M // BM),
        in_specs=[
            pl.BlockSpec((1, three, N), lambda b, j: (b, 0, 0)),
            pl.BlockSpec((1, three, BM), lambda b, j: (b, 0, j)),
        ],
        out_specs=pl.BlockSpec((1, BM, N), lambda b, j: (b, j, 0)),
        out_shape=jax.ShapeDtypeStruct((B, M, N), jnp.float32),
    )(points, new_points)
    dist_flat = dist_t.reshape(B * M, N)

    QT = B * M
    mesh = plsc.VectorSubcoreMesh(core_axis_name="c", subcore_axis_name="s")
    idx_flat = pl.kernel(
        functools.partial(_sctopk_body, n=N, m=M, qpw=QT // NW),
        out_type=jax.ShapeDtypeStruct((QT * S,), jnp.int32),
        mesh=mesh,
        scratch_types=[
            pltpu.VMEM((2 * N,), jnp.float32),
            pltpu.VMEM((N // 16,), jnp.float32),
            pltpu.VMEM((S,), jnp.int32),
            pltpu.SemaphoreType.DMA,
            pltpu.SemaphoreType.DMA,
        ],
        compiler_params=pltpu.CompilerParams(use_tc_tiling_on_sc=False),
    )(dist_flat)

    RT = B * M * S
    rows_per_w = RT // NW
    chunk = 128
    rows = pl.kernel(
        functools.partial(_gather_body, rows_per_w=rows_per_w, chunk=chunk),
        out_type=jax.ShapeDtypeStruct((RT, D), jnp.float32),
        mesh=mesh,
        scratch_types=[
            pltpu.VMEM((chunk,), jnp.int32),
            pltpu.VMEM((chunk, D), jnp.float32),
            pltpu.SemaphoreType.DMA,
        ],
        compiler_params=pltpu.CompilerParams(use_tc_tiling_on_sc=False),
    )(table, idx_flat)

    BMC = 128
    BMS = BMC * S
    qrep = jnp.broadcast_to(new_points[:, :, :, None],
                            (B, three, M, S)).reshape(B, three, M * S)
    rows3 = rows.reshape(B, M * S, D)
    out = pl.pallas_call(
        _finish_body,
        grid=(B, (M * S) // BMS),
        in_specs=[
            pl.BlockSpec((1, BMS, D), lambda b, j: (b, j, 0)),
            pl.BlockSpec((1, three, BMS), lambda b, j: (b, 0, j)),
        ],
        out_specs=pl.BlockSpec((1, C + three, BMS), lambda b, j: (b, 0, j)),
        out_shape=jax.ShapeDtypeStruct((B, C + three, M * S), jnp.float32),
    )(rows3, qrep)
    return out.reshape(B, C + three, M, S)


# speedup vs baseline: 11.3477x; 1.0777x over previous
import functools

import jax
import jax.numpy as jnp
from jax import lax
from jax.experimental import pallas as pl
from jax.experimental.pallas import tpu as pltpu
from jax.experimental.pallas import tpu_sc as plsc

S = 32
D = 144
NW = 32
NC = 2
L = 16


def _table_body(f_ref, p_ref, o_ref):
    f = f_ref[0]
    p = p_ref[0]
    o_ref[0, :, 0:128] = f.T
    o_ref[0, :, 128:131] = p.T


def _dist_body(p_ref, q_ref, o_ref, *, n, nc):
    q = q_ref[0]
    yy = jnp.sum(q * q, axis=0)

    def chunk(i, c):
        pc = p_ref[0, :, pl.ds(i * nc, nc)]
        inner = lax.dot_general(pc, q, (((0,), (0,)), ((), ())),
                                preferred_element_type=jnp.float32)
        xx = jnp.sum(pc * pc, axis=0)
        d = xx[:, None] - 2.0 * inner + yy[None, :]
        o_ref[0, :, pl.ds(i * nc, nc)] = d.T
        return c

    lax.fori_loop(0, n // nc, chunk, 0, unroll=False)


def _xreduce(v, op):
    vals = [v[i] for i in range(L)]
    while len(vals) > 1:
        vals = [op(vals[i], vals[i + 1]) if i + 1 < len(vals) else vals[i]
                for i in range(0, len(vals), 2)]
    return vals[0]


def _sctopk_body(dist_ref, idx_ref, row2, smin_ref, obuf, sem0, sem1,
                 *, n, m, qpw):
    wid = lax.axis_index("s") * NC + lax.axis_index("c")
    qbase = wid * qpw
    inf = jnp.float32(jnp.inf)
    nch = n // L

    iot0 = lax.iota(jnp.int32, L)
    inf16 = jnp.full((L,), inf, jnp.float32)
    big = jnp.int32(1 << 30)
    big16 = jnp.full((L,), big, jnp.int32)
    G = 16
    nst = nch // G

    pltpu.async_copy(dist_ref.at[qbase], row2.at[pl.ds(0, n)], sem0)

    def per_query(j, carry0):
        qid = qbase + j
        b = qid // m
        par = (j & 1) * n
        nxt = ((j + 1) & 1) * n

        @pl.when(((j & 1) == 0) & (j + 1 < qpw))
        def _():
            pltpu.async_copy(dist_ref.at[qid + 1], row2.at[pl.ds(nxt, n)],
                             sem1)

        @pl.when(((j & 1) == 1) & (j + 1 < qpw))
        def _():
            pltpu.async_copy(dist_ref.at[qid + 1], row2.at[pl.ds(nxt, n)],
                             sem0)

        @pl.when((j & 1) == 0)
        def _():
            pltpu.make_async_copy(dist_ref.at[qid], row2.at[pl.ds(par, n)],
                                  sem0).wait()

        @pl.when((j & 1) == 1)
        def _():
            pltpu.make_async_copy(dist_ref.at[qid], row2.at[pl.ds(par, n)],
                                  sem1).wait()

        def p1(s, c):
            def inner(i, acc):
                return jnp.minimum(acc, row2[pl.ds(par + (s * G + i) * L, L)])
            smin_ref[pl.ds(s * L, L)] = lax.fori_loop(0, G, inner, inf16,
                                                      unroll=16)
            return c

        lax.fori_loop(0, nst, p1, 0, unroll=4)

        def extract(si, carry):
            def l12(s, c):
                macc, sarg = c
                v = smin_ref[pl.ds(s * L, L)]
                upd = v < macc
                sarg = jnp.where(upd, s, sarg)
                macc = jnp.minimum(macc, v)
                return macc, sarg

            macc, sarg = lax.fori_loop(0, nst, l12, (inf16, big16), unroll=8)
            gmin = _xreduce(macc, jnp.minimum)
            sstar = _xreduce(jnp.where(macc == gmin, sarg, big16),
                             jnp.minimum)

            def l34(i, c):
                nmin, m1, m2 = c
                base_n = (sstar * G + i) * L
                v = row2[pl.ds(par + base_n, L)]
                eq = v == gmin
                nmin = jnp.minimum(nmin, jnp.where(eq, base_n + iot0, big))
                m2 = jnp.minimum(m2, jnp.maximum(m1, v))
                m1 = jnp.minimum(m1, v)
                return nmin, m1, m2

            nminv, m1, m2 = lax.fori_loop(0, G, l34, (big16, inf16, inf16),
                                          unroll=16)
            nsel = _xreduce(nminv, jnp.minimum)
            lane_sel = iot0 == (nsel & (L - 1))

            cbase = (nsel >> 4) * L
            v = row2[pl.ds(par + cbase, L)]
            row2[pl.ds(par + cbase, L)] = jnp.where(lane_sel, inf, v)

            smin_ref[pl.ds(sstar * L, L)] = jnp.where(lane_sel, m2, m1)

            ob = obuf[pl.ds((si >> 4) * L, L)]
            obuf[pl.ds((si >> 4) * L, L)] = jnp.where(
                iot0 == (si & (L - 1)), b * n + nsel, ob)
            return carry

        lax.fori_loop(0, S, extract, 0, unroll=False)
        pltpu.sync_copy(obuf, idx_ref.at[pl.ds(qid * S, S)])
        return carry0

    lax.fori_loop(0, qpw, per_query, 0, unroll=False)


def _gather_body(tab_ref, idx_ref, out_ref, idx_v, rows_v, sem,
                 *, rows_per_w, chunk):
    wid = lax.axis_index("s") * NC + lax.axis_index("c")
    base = wid * rows_per_w

    def step(i, carry):
        off = base + i * chunk
        pltpu.sync_copy(idx_ref.at[pl.ds(off, chunk)], idx_v)
        pltpu.async_copy(tab_ref.at[idx_v], rows_v, sem).wait()
        pltpu.sync_copy(rows_v, out_ref.at[pl.ds(off, chunk)])
        return carry

    lax.fori_loop(0, rows_per_w // chunk, step, 0)


def _finish_body(r_ref, q_ref, o_ref):
    t = r_ref[0].T
    o_ref[0, 0:3, :] = t[128:131, :] - q_ref[0]
    o_ref[0, 3:131, :] = t[0:128, :]


def kernel(points, new_points, features):
    B, three, N = points.shape
    M = new_points.shape[2]
    C = features.shape[1]

    NB = 1024
    table = pl.pallas_call(
        _table_body,
        grid=(B, N // NB),
        in_specs=[
            pl.BlockSpec((1, C, NB), lambda b, j: (b, 0, j)),
            pl.BlockSpec((1, three, NB), lambda b, j: (b, 0, j)),
        ],
        out_specs=pl.BlockSpec((1, NB, D), lambda b, j: (b, j, 0)),
        out_shape=jax.ShapeDtypeStruct((B, N, D), jnp.float32),
    )(features, points)
    table = table.reshape(B * N, D)

    BM = 256
    NCHUNK = 256
    dist_t = pl.pallas_call(
        functools.partial(_dist_body, n=N, nc=NCHUNK),
        grid=(B, M // BM),
        in_specs=[
            pl.BlockSpec((1, three, N), lambda b, j: (b, 0, 0)),
            pl.BlockSpec((1, three, BM), lambda b, j: (b, 0, j)),
        ],
        out_specs=pl.BlockSpec((1, BM, N), lambda b, j: (b, j, 0)),
        out_shape=jax.ShapeDtypeStruct((B, M, N), jnp.float32),
    )(points, new_points)
    dist_flat = dist_t.reshape(B * M, N)

    QT = B * M
    mesh = plsc.VectorSubcoreMesh(core_axis_name="c", subcore_axis_name="s")
    idx_flat = pl.kernel(
        functools.partial(_sctopk_body, n=N, m=M, qpw=QT // NW),
        out_type=jax.ShapeDtypeStruct((QT * S,), jnp.int32),
        mesh=mesh,
        scratch_types=[
            pltpu.VMEM((2 * N,), jnp.float32),
            pltpu.VMEM((N // 16,), jnp.float32),
            pltpu.VMEM((S,), jnp.int32),
            pltpu.SemaphoreType.DMA,
            pltpu.SemaphoreType.DMA,
        ],
    )(dist_flat)

    RT = B * M * S
    rows_per_w = RT // NW
    chunk = 128
    rows = pl.kernel(
        functools.partial(_gather_body, rows_per_w=rows_per_w, chunk=chunk),
        out_type=jax.ShapeDtypeStruct((RT, D), jnp.float32),
        mesh=mesh,
        scratch_types=[
            pltpu.VMEM((chunk,), jnp.int32),
            pltpu.VMEM((chunk, D), jnp.float32),
            pltpu.SemaphoreType.DMA,
        ],
        compiler_params=pltpu.CompilerParams(use_tc_tiling_on_sc=False),
    )(table, idx_flat)

    BMC = 128
    BMS = BMC * S
    qrep = jnp.broadcast_to(new_points[:, :, :, None],
                            (B, three, M, S)).reshape(B, three, M * S)
    rows3 = rows.reshape(B, M * S, D)
    out = pl.pallas_call(
        _finish_body,
        grid=(B, (M * S) // BMS),
        in_specs=[
            pl.BlockSpec((1, BMS, D), lambda b, j: (b, j, 0)),
            pl.BlockSpec((1, three, BMS), lambda b, j: (b, 0, j)),
        ],
        out_specs=pl.BlockSpec((1, C + three, BMS), lambda b, j: (b, 0, j)),
        out_shape=jax.ShapeDtypeStruct((B, C + three, M * S), jnp.float32),
    )(rows3, qrep)
    return out.reshape(B, C + three, M, S)
